# baseline (speedup 1.0000x reference)
import functools

import jax
import jax.numpy as jnp
from jax import lax
from jax.experimental import pallas as pl
from jax.experimental.pallas import tpu as pltpu
from jax.experimental.pallas import tpu_sc as plsc

B = 4
T = 2048
D = 768
L = 16

_info = plsc.get_sparse_core_info()
NC, NS = _info.num_cores, _info.num_subcores
NW = NC * NS
TPW = T // NW
HALF = TPW // 4
NCHUNK = 4 * B
NBUF = 3


def _body(ids_hbm, wte_hbm, wpe_hbm, out_hbm,
          idx_v, ident_v, pos_v, tok0, tok1, tok2,
          isem, psem, g0, g1, g2, o0, o1, o2):
    wid = lax.axis_index("s") * NC + lax.axis_index("c")
    sid = lax.axis_index("s")
    t0 = wid * TPW
    toks = [tok0, tok1, tok2]
    gsems = [g0, g1, g2]
    osems = [o0, o1, o2]

    id_cps = [pltpu.async_copy(ids_hbm.at[pl.ds(b * T + t0, TPW)],
                               idx_v.at[b], isem)
              for b in range(B)]
    pos_cp = pltpu.async_copy(wpe_hbm.at[pl.ds(t0, TPW)], pos_v, psem)
    for q in range(HALF // L):
        ident_v[pl.ds(q * L, L)] = lax.iota(jnp.int32, L) + q * L
    for cp in id_cps:
        cp.wait()
    pos_cp.wait()

    def gather(hc, tok, sem):
        b, h = hc // 4, hc % 4
        idx = idx_v.at[b, pl.ds(h * HALF, HALF)]
        return pltpu.async_copy(wte_hbm.at[idx], tok, sem)

    pend_g = [None] * NBUF
    pend_o = [None] * NBUF
    pend_g[0] = gather(0, toks[0], gsems[0])

    for hc in range(NCHUNK):
        s = hc % NBUF
        if hc + 1 < NCHUNK:
            sn = (hc + 1) % NBUF
            if pend_o[sn] is not None:
                pend_o[sn].wait()
            pend_g[sn] = gather(hc + 1, toks[sn], gsems[sn])
        pend_g[s].wait()

        tok = toks[s]
        prow0 = (hc % 4) * HALF

        def add_row(i, acc):
            for j in range(D // L):
                sl = pl.ds(j * L, L)
                acc = acc + pos_v[prow0 + i, sl]
            return acc

        acc = lax.fori_loop(0, HALF, add_row,
                            jnp.zeros((L,), jnp.float32))
        tok[0, pl.ds(0, L)] = acc

        b, h = hc // 4, hc % 4
        base = b * T + t0 + h * HALF
        pend_o[s] = pltpu.async_copy(toks[s],
                                     out_hbm.at[pl.ds(base, HALF)], osems[s])
    for cp in pend_o:
        if cp is not None:
            cp.wait()


@jax.jit
def kernel(ids, wte, wpe):
    mesh = plsc.VectorSubcoreMesh(core_axis_name="c", subcore_axis_name="s")
    run = functools.partial(
        pl.kernel,
        mesh=mesh,
        out_type=jax.ShapeDtypeStruct((B * T, D), jnp.float32),
        scratch_types=[
            pltpu.VMEM((B, TPW), jnp.int32),
            pltpu.VMEM((HALF,), jnp.int32),
            pltpu.VMEM((TPW, D), jnp.float32),
            pltpu.VMEM((HALF, D), jnp.float32),
            pltpu.VMEM((HALF, D), jnp.float32),
            pltpu.VMEM((HALF, D), jnp.float32),
        ] + [pltpu.SemaphoreType.DMA] * 8,
    )(_body)
    out = run(ids.reshape(B * T).astype(jnp.int32), wte, wpe)
    return out.reshape(B, T, D)


# speedup vs baseline: 1.0773x; 1.0773x over previous
import functools

import jax
import jax.numpy as jnp
from jax import lax
from jax.experimental import pallas as pl
from jax.experimental.pallas import tpu as pltpu
from jax.experimental.pallas import tpu_sc as plsc

B = 4
T = 2048
D = 768
L = 16
NVEC = D // L

_info = plsc.get_sparse_core_info()
NC, NS = _info.num_cores, _info.num_subcores
NW = NC * NS
TPW = T // NW
Q = 8
NGRP = TPW // Q
NBUF = 3


def _body(ids_hbm, wte_hbm, wpe_hbm, out_hbm,
          idx_v, pos_v, *rest):
    toks = [[rest[s * B + b] for b in range(B)] for s in range(NBUF)]
    isem, psem = rest[NBUF * B], rest[NBUF * B + 1]
    gsems = rest[NBUF * B + 2:NBUF * B + 2 + NBUF]
    osems = rest[NBUF * B + 2 + NBUF:NBUF * B + 2 + 2 * NBUF]

    wid = lax.axis_index("s") * NC + lax.axis_index("c")
    t0 = wid * TPW

    id_cps = [pltpu.async_copy(ids_hbm.at[pl.ds(b * T + t0, TPW)],
                               idx_v.at[b], isem)
              for b in range(B)]
    pos_cp = pltpu.async_copy(wpe_hbm.at[pl.ds(t0, TPW)], pos_v, psem)
    for cp in id_cps:
        cp.wait()

    def gathers(g, s):
        return [pltpu.async_copy(
                    wte_hbm.at[idx_v.at[b, pl.ds(g * Q, Q)]],
                    toks[s][b], gsems[s])
                for b in range(B)]

    pend_g = [None] * NBUF
    pend_o = [None] * NBUF
    pend_g[0] = gathers(0, 0)

    for g in range(NGRP):
        s = g % NBUF
        if g + 1 < NGRP:
            sn = (g + 1) % NBUF
            if pend_o[sn] is not None:
                for cp in pend_o[sn]:
                    cp.wait()
            pend_g[sn] = gathers(g + 1, sn)
        for cp in pend_g[s]:
            cp.wait()
        if g == 0:
            pos_cp.wait()

        tok_b = toks[s]
        grow0 = g * Q

        def add_row(r, carry):
            for j in range(NVEC):
                sl = pl.ds(j * L, L)
                v = pos_v[grow0 + r, sl]
                for b in range(B):
                    tok_b[b][r, sl] = tok_b[b][r, sl] + v
            return carry

        lax.fori_loop(0, Q, add_row, 0)

        pend_o[s] = [pltpu.async_copy(
                         tok_b[b],
                         out_hbm.at[pl.ds(b * T + t0 + grow0, Q)],
                         osems[s])
                     for b in range(B)]
    for cps in pend_o:
        if cps is not None:
            for cp in cps:
                cp.wait()


@jax.jit
def kernel(ids, wte, wpe):
    mesh = plsc.VectorSubcoreMesh(core_axis_name="c", subcore_axis_name="s")
    run = functools.partial(
        pl.kernel,
        mesh=mesh,
        out_type=jax.ShapeDtypeStruct((B * T, D), jnp.float32),
        scratch_types=(
            [pltpu.VMEM((B, TPW), jnp.int32),
             pltpu.VMEM((TPW, D), jnp.float32)]
            + [pltpu.VMEM((Q, D), jnp.float32) for _ in range(NBUF * B)]
            + [pltpu.SemaphoreType.DMA] * (2 + 2 * NBUF)
        ),
    )(_body)
    out = run(ids.reshape(B * T).astype(jnp.int32), wte, wpe)
    return out.reshape(B, T, D)


# speedup vs baseline: 1.0799x; 1.0025x over previous
import functools

import jax
import jax.numpy as jnp
from jax import lax
from jax.experimental import pallas as pl
from jax.experimental.pallas import tpu as pltpu
from jax.experimental.pallas import tpu_sc as plsc

B = 4
T = 2048
D = 768
L = 16
NVEC = D // L

_info = plsc.get_sparse_core_info()
NC, NS = _info.num_cores, _info.num_subcores
NW = NC * NS
TPW = T // NW
Q = 8
NGRP = TPW // Q
NBUF = 3


def _body(ids_hbm, wte_hbm, wpe_hbm, out_hbm,
          idx_v, pos_v, *rest):
    toks = [[rest[s * B + b] for b in range(B)] for s in range(NBUF)]
    isem, psem = rest[NBUF * B], rest[NBUF * B + 1]
    gsems = rest[NBUF * B + 2:NBUF * B + 2 + NBUF]
    osems = rest[NBUF * B + 2 + NBUF:NBUF * B + 2 + 2 * NBUF]

    wid = lax.axis_index("s") * NC + lax.axis_index("c")
    t0 = wid * TPW

    id_cps = [pltpu.async_copy(ids_hbm.at[pl.ds(b * T + t0, TPW)],
                               idx_v.at[b], isem)
              for b in range(B)]
    pos_cp = pltpu.async_copy(wpe_hbm.at[pl.ds(t0, TPW)], pos_v, psem)
    for cp in id_cps:
        cp.wait()

    def gathers(g, s):
        return [pltpu.async_copy(
                    wte_hbm.at[idx_v.at[b, pl.ds(g * Q, Q)]],
                    toks[s][b], gsems[s])
                for b in range(B)]

    pend_g = [None] * NBUF
    pend_o = [None] * NBUF
    pend_g[0] = gathers(0, 0)

    for g in range(NGRP):
        s = g % NBUF
        if g + 1 < NGRP:
            sn = (g + 1) % NBUF
            if pend_o[sn] is not None:
                for cp in pend_o[sn]:
                    cp.wait()
            pend_g[sn] = gathers(g + 1, sn)
        for cp in pend_g[s]:
            cp.wait()
        if g == 0:
            pos_cp.wait()

        tok_b = toks[s]
        grow0 = g * Q

        def add_row(r, carry):
            for j in range(NVEC):
                sl = pl.ds(j * L, L)
                v = pos_v[grow0 + r, sl]
                for b in range(B):
                    plsc.addupdate(tok_b[b].at[r, sl], v)
            return carry

        lax.fori_loop(0, Q, add_row, 0)

        pend_o[s] = [pltpu.async_copy(
                         tok_b[b],
                         out_hbm.at[pl.ds(b * T + t0 + grow0, Q)],
                         osems[s])
                     for b in range(B)]
    for cps in pend_o:
        if cps is not None:
            for cp in cps:
                cp.wait()


@jax.jit
def kernel(ids, wte, wpe):
    mesh = plsc.VectorSubcoreMesh(core_axis_name="c", subcore_axis_name="s")
    run = functools.partial(
        pl.kernel,
        mesh=mesh,
        out_type=jax.ShapeDtypeStruct((B * T, D), jnp.float32),
        scratch_types=(
            [pltpu.VMEM((B, TPW), jnp.int32),
             pltpu.VMEM((TPW, D), jnp.float32)]
            + [pltpu.VMEM((Q, D), jnp.float32) for _ in range(NBUF * B)]
            + [pltpu.SemaphoreType.DMA] * (2 + 2 * NBUF)
        ),
    )(_body)
    out = run(ids.reshape(B * T).astype(jnp.int32), wte, wpe)
    return out.reshape(B, T, D)


# speedup vs baseline: 1.0860x; 1.0056x over previous
import functools

import jax
import jax.numpy as jnp
from jax import lax
from jax.experimental import pallas as pl
from jax.experimental.pallas import tpu as pltpu
from jax.experimental.pallas import tpu_sc as plsc

B = 4
T = 2048
D = 768
L = 16
NVEC = D // L

_info = plsc.get_sparse_core_info()
NC, NS = _info.num_cores, _info.num_subcores
NW = NC * NS
TPW = T // NW
Q = 8
NGRP = TPW // Q
GR = B * Q
NBUF = 3


def _body(ids_hbm, wte_hbm, wpe_hbm, out_hbm,
          idx_v, pos_v, tok0, tok1, tok2,
          isem, psem, g0, g1, g2, o0, o1, o2):
    toks = [tok0, tok1, tok2]
    gsems = [g0, g1, g2]
    osems = [o0, o1, o2]

    wid = lax.axis_index("s") * NC + lax.axis_index("c")
    t0 = wid * TPW

    id_cp = pltpu.async_copy(ids_hbm.at[pl.ds(wid * (NGRP * GR), NGRP * GR)],
                             idx_v, isem)
    pos_cp = pltpu.async_copy(wpe_hbm.at[pl.ds(t0, TPW)], pos_v, psem)
    id_cp.wait()

    def gather(g, s):
        return pltpu.async_copy(wte_hbm.at[idx_v.at[pl.ds(g * GR, GR)]],
                                toks[s], gsems[s])

    pend_g = [None] * NBUF
    pend_o = [None] * NBUF
    pend_g[0] = gather(0, 0)

    for g in range(NGRP):
        s = g % NBUF
        if g + 1 < NGRP:
            sn = (g + 1) % NBUF
            if pend_o[sn] is not None:
                for cp in pend_o[sn]:
                    cp.wait()
            pend_g[sn] = gather(g + 1, sn)
        pend_g[s].wait()
        if g == 0:
            pos_cp.wait()

        tok = toks[s]
        grow0 = g * Q

        def add_row(r, carry):
            for j in range(NVEC):
                sl = pl.ds(j * L, L)
                v = pos_v[grow0 + r, sl]
                for b in range(B):
                    plsc.addupdate(tok.at[b * Q + r, sl], v)
            return carry

        lax.fori_loop(0, Q, add_row, 0)

        pend_o[s] = [pltpu.async_copy(
                         tok.at[pl.ds(b * Q, Q)],
                         out_hbm.at[pl.ds(b * T + t0 + grow0, Q)],
                         osems[s])
                     for b in range(B)]
    for cps in pend_o:
        if cps is not None:
            for cp in cps:
                cp.wait()


@jax.jit
def kernel(ids, wte, wpe):
    ids_p = (ids.astype(jnp.int32)
             .reshape(B, NW, NGRP, Q)
             .transpose(1, 2, 0, 3)
             .reshape(NW * NGRP * GR))
    mesh = plsc.VectorSubcoreMesh(core_axis_name="c", subcore_axis_name="s")
    run = functools.partial(
        pl.kernel,
        mesh=mesh,
        out_type=jax.ShapeDtypeStruct((B * T, D), jnp.float32),
        scratch_types=[
            pltpu.VMEM((NGRP * GR,), jnp.int32),
            pltpu.VMEM((TPW, D), jnp.float32),
            pltpu.VMEM((GR, D), jnp.float32),
            pltpu.VMEM((GR, D), jnp.float32),
            pltpu.VMEM((GR, D), jnp.float32),
        ] + [pltpu.SemaphoreType.DMA] * 8,
    )(_body)
    out = run(ids_p, wte, wpe)
    return out.reshape(B, T, D)
